# Initial kernel scaffold; baseline (speedup 1.0000x reference)
import jax
import jax.numpy as jnp
from jax.experimental import pallas as pl


def kernel(inputs, gm):
    raise NotImplementedError("write your pallas kernel here")


# speedup vs baseline: 20.0394x; 20.0394x over previous
import functools

import jax
import jax.numpy as jnp
from jax.experimental import pallas as pl
from jax.experimental.pallas import tpu as pltpu

K = 64
N = 128
LLR_MAX = 100.0
TAU = 3e-6
BE = 64


def _body(llr_ref, gm_ref, out_ref, state_ref, lv_ref):
    llr = jnp.clip(llr_ref[...], -LLR_MAX, LLR_MAX)
    a = jnp.abs(llr)
    gm = gm_ref[...]
    state_ref[...] = jnp.broadcast_to(gm[:, None, :], (K, BE, N))

    lane = jax.lax.broadcasted_iota(jnp.int32, (BE, N), 1)

    def step(i, _):
        row = state_ref[pl.ds(i, 1)][0]
        score = jnp.where(row > 0.5, a, -1.0)
        m = jnp.max(score, axis=-1, keepdims=True)
        hit = score >= m
        idx = jnp.min(jnp.where(hit, lane, N), axis=-1, keepdims=True)
        oh = (lane == idx).astype(jnp.float32)
        lv = jnp.sum(oh * llr, axis=-1)
        lv_ref[pl.ds(i, 1)] = lv.reshape(1, BE)
        st = state_ref[...]
        hb = jnp.sum(st * oh[None, :, :], axis=-1)
        ri = jax.lax.broadcasted_iota(jnp.int32, (K, BE), 0)
        hb = jnp.where(ri == i, 0.0, hb)
        upd = hb[:, :, None] * row[None, :, :]
        state_ref[...] = st + upd - 2.0 * st * upd
        return 0

    jax.lax.fori_loop(0, K, step, 0)

    st = state_ref[...]
    u = (lv_ref[...] > 0.0).astype(jnp.float32)
    csum = jnp.sum(u[:, :, None] * st, axis=0)
    c = csum - 2.0 * jnp.floor(csum * 0.5)
    v = (1.0 - 2.0 * c) * llr
    delta = jnp.sum(st * v[None, :, :], axis=-1)
    dmax = jnp.max(delta, axis=0, keepdims=True)
    okm = delta >= dmax - TAU
    ri = jax.lax.broadcasted_iota(jnp.int32, (K, BE), 0)
    isel = jnp.min(jnp.where(okm, ri, K), axis=0, keepdims=True)
    ohrow = (ri == isel).astype(jnp.float32)
    dsel = jnp.sum(ohrow * delta, axis=0)
    erow = jnp.sum(ohrow[:, :, None] * st, axis=0)
    flip = (dsel > TAU).astype(jnp.float32).reshape(BE, 1)
    e = flip * erow
    out_ref[...] = c + e - 2.0 * c * e


@jax.jit
def kernel(inputs, gm):
    shape = inputs.shape
    llr = inputs.reshape(-1, N).astype(jnp.float32)
    bs = llr.shape[0]
    grid = bs // BE
    out = pl.pallas_call(
        _body,
        grid=(grid,),
        in_specs=[
            pl.BlockSpec((BE, N), lambda i: (i, 0)),
            pl.BlockSpec((K, N), lambda i: (0, 0)),
        ],
        out_specs=pl.BlockSpec((BE, N), lambda i: (i, 0)),
        out_shape=jax.ShapeDtypeStruct((bs, N), jnp.float32),
        scratch_shapes=[
            pltpu.VMEM((K, BE, N), jnp.float32),
            pltpu.VMEM((K, BE), jnp.float32),
        ],
    )(llr, gm.astype(jnp.float32))
    return out.reshape(shape)


# speedup vs baseline: 148.3667x; 7.4037x over previous
import functools

import jax
import jax.numpy as jnp
from jax import lax
from jax.experimental import pallas as pl
from jax.experimental.pallas import tpu as pltpu
from jax.experimental.pallas import tpu_sc as plsc

K = 64
N = 128
NWORD = N // 32
LLR_MAX = 100.0
TAU = 3e-6
NC, NS, L = 2, 16, 16
NW = NC * NS
BS = 512
EPW = BS // NW


def _worker_id():
    return lax.axis_index("s") * NC + lax.axis_index("c")


def _sc_body(llr_hbm, gml_hbm, out_hbm, llr_v, a_v, st_v, lv_v, d_v, v_v, o_v):
    w = _worker_id()
    lane = lax.broadcasted_iota(jnp.int32, (L,), 0)

    pltpu.sync_copy(llr_hbm.at[w], llr_v)
    pltpu.sync_copy(gml_hbm, st_v)

    def prep(j, _):
        x = jnp.clip(llr_v[pl.ds(j * L, L)], -LLR_MAX, LLR_MAX)
        llr_v[pl.ds(j * L, L)] = x
        a_v[pl.ds(j * L, L)] = jnp.abs(x)
        return 0

    lax.fori_loop(0, N, prep, 0, unroll=8)

    def step(i, _):
        i4 = i * NWORD
        rws = [st_v[pl.ds((i4 + t) * L, L)] for t in range(NWORD)]
        best = jnp.full((L,), -1.0, jnp.float32)
        jsel = jnp.zeros((L,), jnp.int32)
        for t in range(NWORD):
            tw = rws[t]
            for b in range(32):
                j = t * 32 + b
                aj = a_v[j * L:(j + 1) * L]
                m = ((tw & 1) != 0) & (aj > best)
                best = jnp.where(m, aj, best)
                jsel = jnp.where(m, j, jsel)
                tw = lax.shift_right_logical(tw, 1)
        lv_v[pl.ds(i * L, L)] = plsc.load_gather(llr_v, [jsel * L + lane])
        jw = lax.shift_right_logical(jsel, 5)
        jb = jsel & 31

        def rowupd(r, _):
            tw = plsc.load_gather(st_v, [(r * NWORD + jw) * L + lane])
            hb = lax.shift_right_logical(tw, jb) & 1
            nm = jnp.where(r == i, 0, 1)
            msk = -(hb * nm)
            for t in range(NWORD):
                k = (r * NWORD + t) * L
                st_v[pl.ds(k, L)] = st_v[pl.ds(k, L)] ^ (msk & rws[t])
            return 0

        lax.fori_loop(0, K, rowupd, 0)
        return 0

    lax.fori_loop(0, K, step, 0)

    def cacc(i, cw):
        u = (lv_v[pl.ds(i * L, L)] > 0.0).astype(jnp.int32)
        m = -u
        return tuple(cw[t] ^ (m & st_v[pl.ds((i * NWORD + t) * L, L)])
                     for t in range(NWORD))

    cws = lax.fori_loop(0, K, cacc,
                        tuple(jnp.zeros((L,), jnp.int32) for _ in range(NWORD)))

    for t in range(NWORD):
        tw = cws[t]
        for b in range(32):
            j = t * 32 + b
            cb = (tw & 1).astype(jnp.float32)
            x = llr_v[j * L:(j + 1) * L]
            v_v[j * L:(j + 1) * L] = x - 2.0 * cb * x
            tw = lax.shift_right_logical(tw, 1)

    def drow(i, _):
        i4 = i * NWORD
        acc = jnp.zeros((L,), jnp.float32)
        for t in range(NWORD):
            tw = st_v[pl.ds((i4 + t) * L, L)]
            for b in range(32):
                j = t * 32 + b
                acc = acc + (tw & 1).astype(jnp.float32) * v_v[j * L:(j + 1) * L]
                tw = lax.shift_right_logical(tw, 1)
        d_v[pl.ds(i * L, L)] = acc
        return 0

    lax.fori_loop(0, K, drow, 0)

    def dmaxf(i, dm):
        return jnp.maximum(dm, d_v[pl.ds(i * L, L)])

    dmax = lax.fori_loop(0, K, dmaxf, jnp.full((L,), -jnp.inf, jnp.float32))

    def firstsel(i, isel):
        hit = (isel >= K) & (d_v[pl.ds(i * L, L)] >= dmax - TAU)
        return jnp.where(hit, i, isel)

    isel = lax.fori_loop(0, K, firstsel, jnp.full((L,), K, jnp.int32))
    dsel = plsc.load_gather(d_v, [isel * L + lane])
    fm = -(dsel > TAU).astype(jnp.int32)

    ews = [plsc.load_gather(st_v, [(isel * NWORD + t) * L + lane]) & fm
           for t in range(NWORD)]
    for t in range(NWORD):
        ow = cws[t] ^ ews[t]
        for b in range(32):
            j = t * 32 + b
            o_v[j * L:(j + 1) * L] = (ow & 1).astype(jnp.float32)
            ow = lax.shift_right_logical(ow, 1)

    pltpu.sync_copy(o_v, out_hbm.at[w])


def _make_sc_kernel(interpret=False):
    return functools.partial(
        pl.kernel,
        out_type=jax.ShapeDtypeStruct((NW, N * EPW), jnp.float32),
        mesh=plsc.VectorSubcoreMesh(core_axis_name="c", subcore_axis_name="s",
                                    num_cores=NC, num_subcores=NS),
        scratch_types=[
            pltpu.VMEM((N * L,), jnp.float32),
            pltpu.VMEM((N * L,), jnp.float32),
            pltpu.VMEM((K * NWORD * L,), jnp.int32),
            pltpu.VMEM((K * L,), jnp.float32),
            pltpu.VMEM((K * L,), jnp.float32),
            pltpu.VMEM((N * L,), jnp.float32),
            pltpu.VMEM((N * L,), jnp.float32),
        ],
        compiler_params=pltpu.CompilerParams(needs_layout_passes=False),
        interpret=interpret,
    )(_sc_body)


@jax.jit
def kernel(inputs, gm):
    shape = inputs.shape
    llr = inputs.reshape(-1, N).astype(jnp.float32)
    bs = llr.shape[0]
    llr3 = llr.reshape(NW, EPW, N).transpose(0, 2, 1)
    gmi = gm.astype(jnp.int32)
    shifts = jnp.arange(32, dtype=jnp.int32)
    gmb = (gmi.reshape(K, NWORD, 32) << shifts[None, None, :]).sum(
        axis=-1, dtype=jnp.int32)
    gml = jnp.broadcast_to(gmb.reshape(K * NWORD, 1), (K * NWORD, L))
    gml = jnp.asarray(gml, jnp.int32).reshape(K * NWORD * L)
    out3 = _make_sc_kernel()(llr3.reshape(NW, N * EPW), gml)
    out = out3.reshape(NW, N, EPW).transpose(0, 2, 1).reshape(bs, N)
    return out.reshape(shape)


# speedup vs baseline: 150.9445x; 1.0174x over previous
import functools

import jax
import jax.numpy as jnp
from jax import lax
from jax.experimental import pallas as pl
from jax.experimental.pallas import tpu as pltpu
from jax.experimental.pallas import tpu_sc as plsc

K = 64
N = 128
NWORD = N // 32
LLR_MAX = 100.0
TAU = 3e-6
NC, NS, L = 2, 16, 16
NW = NC * NS
BS = 512
EPW = BS // NW


def _worker_id():
    return lax.axis_index("s") * NC + lax.axis_index("c")


def _sc_body(llr_hbm, gml_hbm, out_hbm, llr_v, a_v, st_v, lv_v, d_v, v_v, o_v):
    w = _worker_id()
    lane = lax.broadcasted_iota(jnp.int32, (L,), 0)

    pltpu.sync_copy(llr_hbm.at[w], llr_v)
    pltpu.sync_copy(gml_hbm, st_v)

    def prep(j, _):
        x = jnp.clip(llr_v[pl.ds(j * L, L)], -LLR_MAX, LLR_MAX)
        llr_v[pl.ds(j * L, L)] = x
        a_v[pl.ds(j * L, L)] = jnp.abs(x)
        return 0

    lax.fori_loop(0, N, prep, 0, unroll=8)

    def step(i, _):
        i4 = i * NWORD
        rws = [st_v[pl.ds((i4 + t) * L, L)] for t in range(NWORD)]
        bests, jsels = [], []
        for t in range(NWORD):
            t2 = lax.shift_left(rws[t], 0)
            best = jnp.full((L,), -1.0, jnp.float32)
            jsel = jnp.zeros((L,), jnp.int32)
            for b in range(31, -1, -1):
                j = t * 32 + b
                aj = a_v[j * L:(j + 1) * L]
                m = (t2 < 0) & (aj >= best)
                best = jnp.where(m, aj, best)
                jsel = jnp.where(m, j, jsel)
                t2 = lax.shift_left(t2, 1)
            bests.append(best)
            jsels.append(jsel)
        best, jsel = bests[0], jsels[0]
        for t in range(1, NWORD):
            m = bests[t] > best
            best = jnp.where(m, bests[t], best)
            jsel = jnp.where(m, jsels[t], jsel)
        lv_v[pl.ds(i * L, L)] = plsc.load_gather(llr_v, [jsel * L + lane])
        jw = lax.shift_right_logical(jsel, 5)
        jb = jsel & 31

        def rowupd(r, _):
            tw = plsc.load_gather(st_v, [(r * NWORD + jw) * L + lane])
            msk = -(lax.shift_right_logical(tw, jb) & 1)
            for t in range(NWORD):
                k = (r * NWORD + t) * L
                st_v[pl.ds(k, L)] = st_v[pl.ds(k, L)] ^ (msk & rws[t])
            return 0

        lax.fori_loop(0, K, rowupd, 0, unroll=8)
        for t in range(NWORD):
            st_v[pl.ds((i4 + t) * L, L)] = rws[t]
        return 0

    lax.fori_loop(0, K, step, 0)

    def cacc(i, cw):
        u = (lv_v[pl.ds(i * L, L)] > 0.0).astype(jnp.int32)
        m = -u
        return tuple(cw[t] ^ (m & st_v[pl.ds((i * NWORD + t) * L, L)])
                     for t in range(NWORD))

    cws = lax.fori_loop(0, K, cacc,
                        tuple(jnp.zeros((L,), jnp.int32) for _ in range(NWORD)))

    for t in range(NWORD):
        tw = cws[t]
        for b in range(32):
            j = t * 32 + b
            cb = (tw & 1).astype(jnp.float32)
            x = llr_v[j * L:(j + 1) * L]
            v_v[j * L:(j + 1) * L] = x - 2.0 * cb * x
            tw = lax.shift_right_logical(tw, 1)

    zero = jnp.zeros((L,), jnp.float32)

    def drow(i, _):
        i4 = i * NWORD
        accs = []
        for t in range(NWORD):
            t2 = st_v[pl.ds((i4 + t) * L, L)]
            acc = zero
            for b in range(31, -1, -1):
                j = t * 32 + b
                acc = acc + jnp.where(t2 < 0, v_v[j * L:(j + 1) * L], 0.0)
                t2 = lax.shift_left(t2, 1)
            accs.append(acc)
        d_v[pl.ds(i * L, L)] = (accs[0] + accs[1]) + (accs[2] + accs[3])
        return 0

    lax.fori_loop(0, K, drow, 0)

    def dmaxf(i, dm):
        return jnp.maximum(dm, d_v[pl.ds(i * L, L)])

    dmax = lax.fori_loop(0, K, dmaxf, jnp.full((L,), -jnp.inf, jnp.float32))

    def firstsel(i, isel):
        hit = (isel >= K) & (d_v[pl.ds(i * L, L)] >= dmax - TAU)
        return jnp.where(hit, i, isel)

    isel = lax.fori_loop(0, K, firstsel, jnp.full((L,), K, jnp.int32))
    dsel = plsc.load_gather(d_v, [isel * L + lane])
    fm = -(dsel > TAU).astype(jnp.int32)

    ews = [plsc.load_gather(st_v, [(isel * NWORD + t) * L + lane]) & fm
           for t in range(NWORD)]
    for t in range(NWORD):
        ow = cws[t] ^ ews[t]
        for b in range(32):
            j = t * 32 + b
            o_v[j * L:(j + 1) * L] = (ow & 1).astype(jnp.float32)
            ow = lax.shift_right_logical(ow, 1)

    pltpu.sync_copy(o_v, out_hbm.at[w])


def _make_sc_kernel(interpret=False):
    return functools.partial(
        pl.kernel,
        out_type=jax.ShapeDtypeStruct((NW, N * EPW), jnp.float32),
        mesh=plsc.VectorSubcoreMesh(core_axis_name="c", subcore_axis_name="s",
                                    num_cores=NC, num_subcores=NS),
        scratch_types=[
            pltpu.VMEM((N * L,), jnp.float32),
            pltpu.VMEM((N * L,), jnp.float32),
            pltpu.VMEM((K * NWORD * L,), jnp.int32),
            pltpu.VMEM((K * L,), jnp.float32),
            pltpu.VMEM((K * L,), jnp.float32),
            pltpu.VMEM((N * L,), jnp.float32),
            pltpu.VMEM((N * L,), jnp.float32),
        ],
        compiler_params=pltpu.CompilerParams(needs_layout_passes=False),
        interpret=interpret,
    )(_sc_body)


@jax.jit
def kernel(inputs, gm):
    shape = inputs.shape
    llr = inputs.reshape(-1, N).astype(jnp.float32)
    bs = llr.shape[0]
    llr3 = llr.reshape(NW, EPW, N).transpose(0, 2, 1)
    gmi = gm.astype(jnp.int32)
    shifts = jnp.arange(32, dtype=jnp.int32)
    gmb = (gmi.reshape(K, NWORD, 32) << shifts[None, None, :]).sum(
        axis=-1, dtype=jnp.int32)
    gml = jnp.broadcast_to(gmb.reshape(K * NWORD, 1), (K * NWORD, L))
    gml = jnp.asarray(gml, jnp.int32).reshape(K * NWORD * L)
    out3 = _make_sc_kernel()(llr3.reshape(NW, N * EPW), gml)
    out = out3.reshape(NW, N, EPW).transpose(0, 2, 1).reshape(bs, N)
    return out.reshape(shape)


# speedup vs baseline: 261.1100x; 1.7298x over previous
import functools

import jax
import jax.numpy as jnp
from jax import lax
from jax.experimental import pallas as pl
from jax.experimental.pallas import tpu as pltpu
from jax.experimental.pallas import tpu_sc as plsc

K = 64
N = 128
NWORD = N // 32
LLR_MAX = 100.0
TAU = 3e-6
NC, NS, L = 2, 16, 16
NW = NC * NS
BS = 512
EPW = BS // NW


def _worker_id():
    return lax.axis_index("s") * NC + lax.axis_index("c")


def _sc_body(llr_hbm, gml_hbm, out_hbm, llr_v, a_v, st_v, lv_v, d_v, v_v, o_v):
    w = _worker_id()
    lane = lax.broadcasted_iota(jnp.int32, (L,), 0)

    pltpu.sync_copy(llr_hbm.at[w], llr_v)
    pltpu.sync_copy(gml_hbm, st_v)

    def prep(j, _):
        x = jnp.clip(llr_v[pl.ds(j * L, L)], -LLR_MAX, LLR_MAX)
        llr_v[pl.ds(j * L, L)] = x
        a_v[pl.ds(j * L, L)] = jnp.abs(x)
        return 0

    lax.fori_loop(0, N, prep, 0, unroll=8)

    def step(i, _):
        i4 = i * NWORD
        rws = [st_v[pl.ds((i4 + t) * L, L)] for t in range(NWORD)]
        t2s = list(rws)
        bests = [jnp.full((L,), -1.0, jnp.float32) for _ in range(NWORD)]
        jsels = [jnp.zeros((L,), jnp.int32) for _ in range(NWORD)]
        for b in range(31, -1, -1):
            for t in range(NWORD):
                j = t * 32 + b
                aj = a_v[j * L:(j + 1) * L]
                m = (t2s[t] < 0) & (aj >= bests[t])
                bests[t] = jnp.where(m, aj, bests[t])
                jsels[t] = jnp.where(m, j, jsels[t])
                t2s[t] = lax.shift_left(t2s[t], 1)
        best, jsel = bests[0], jsels[0]
        for t in range(1, NWORD):
            m = bests[t] > best
            best = jnp.where(m, bests[t], best)
            jsel = jnp.where(m, jsels[t], jsel)
        lv_v[pl.ds(i * L, L)] = plsc.load_gather(llr_v, [jsel * L + lane])
        jw = lax.shift_right_logical(jsel, 5)
        jb2 = 31 - (jsel & 31)
        mjw = [jw == t for t in range(1, NWORD)]

        def rowupd(r, _):
            k = r * NWORD * L
            sw = [st_v[pl.ds(k + t * L, L)] for t in range(NWORD)]
            tw = sw[0]
            for t in range(1, NWORD):
                tw = jnp.where(mjw[t - 1], sw[t], tw)
            msk = lax.shift_right_arithmetic(lax.shift_left(tw, jb2), 31)
            for t in range(NWORD):
                st_v[pl.ds(k + t * L, L)] = sw[t] ^ (msk & rws[t])
            return 0

        lax.fori_loop(0, K, rowupd, 0, unroll=8)
        for t in range(NWORD):
            st_v[pl.ds((i4 + t) * L, L)] = rws[t]
        return 0

    lax.fori_loop(0, K, step, 0)

    def cacc(i, cw):
        u = (lv_v[pl.ds(i * L, L)] > 0.0).astype(jnp.int32)
        m = -u
        return tuple(cw[t] ^ (m & st_v[pl.ds((i * NWORD + t) * L, L)])
                     for t in range(NWORD))

    cws = lax.fori_loop(0, K, cacc,
                        tuple(jnp.zeros((L,), jnp.int32) for _ in range(NWORD)))

    for t in range(NWORD):
        tw = cws[t]
        for b in range(32):
            j = t * 32 + b
            cb = (tw & 1).astype(jnp.float32)
            x = llr_v[j * L:(j + 1) * L]
            v_v[j * L:(j + 1) * L] = x - 2.0 * cb * x
            tw = lax.shift_right_logical(tw, 1)

    zero = jnp.zeros((L,), jnp.float32)

    def drow(i, _):
        i4 = i * NWORD
        accs = []
        for t in range(NWORD):
            t2 = st_v[pl.ds((i4 + t) * L, L)]
            acc = zero
            for b in range(31, -1, -1):
                j = t * 32 + b
                acc = acc + jnp.where(t2 < 0, v_v[j * L:(j + 1) * L], 0.0)
                t2 = lax.shift_left(t2, 1)
            accs.append(acc)
        d_v[pl.ds(i * L, L)] = (accs[0] + accs[1]) + (accs[2] + accs[3])
        return 0

    lax.fori_loop(0, K, drow, 0)

    def dmaxf(i, dm):
        return jnp.maximum(dm, d_v[pl.ds(i * L, L)])

    dmax = lax.fori_loop(0, K, dmaxf, jnp.full((L,), -jnp.inf, jnp.float32))

    def firstsel(i, isel):
        hit = (isel >= K) & (d_v[pl.ds(i * L, L)] >= dmax - TAU)
        return jnp.where(hit, i, isel)

    isel = lax.fori_loop(0, K, firstsel, jnp.full((L,), K, jnp.int32))
    dsel = plsc.load_gather(d_v, [isel * L + lane])
    fm = -(dsel > TAU).astype(jnp.int32)

    ews = [plsc.load_gather(st_v, [(isel * NWORD + t) * L + lane]) & fm
           for t in range(NWORD)]
    for t in range(NWORD):
        ow = cws[t] ^ ews[t]
        for b in range(32):
            j = t * 32 + b
            o_v[j * L:(j + 1) * L] = (ow & 1).astype(jnp.float32)
            ow = lax.shift_right_logical(ow, 1)

    pltpu.sync_copy(o_v, out_hbm.at[w])


def _make_sc_kernel(interpret=False):
    return functools.partial(
        pl.kernel,
        out_type=jax.ShapeDtypeStruct((NW, N * EPW), jnp.float32),
        mesh=plsc.VectorSubcoreMesh(core_axis_name="c", subcore_axis_name="s",
                                    num_cores=NC, num_subcores=NS),
        scratch_types=[
            pltpu.VMEM((N * L,), jnp.float32),
            pltpu.VMEM((N * L,), jnp.float32),
            pltpu.VMEM((K * NWORD * L,), jnp.int32),
            pltpu.VMEM((K * L,), jnp.float32),
            pltpu.VMEM((K * L,), jnp.float32),
            pltpu.VMEM((N * L,), jnp.float32),
            pltpu.VMEM((N * L,), jnp.float32),
        ],
        compiler_params=pltpu.CompilerParams(needs_layout_passes=False),
        interpret=interpret,
    )(_sc_body)


@jax.jit
def kernel(inputs, gm):
    shape = inputs.shape
    llr = inputs.reshape(-1, N).astype(jnp.float32)
    bs = llr.shape[0]
    llr3 = llr.reshape(NW, EPW, N).transpose(0, 2, 1)
    gmi = gm.astype(jnp.int32)
    shifts = jnp.arange(32, dtype=jnp.int32)
    gmb = (gmi.reshape(K, NWORD, 32) << shifts[None, None, :]).sum(
        axis=-1, dtype=jnp.int32)
    gml = jnp.broadcast_to(gmb.reshape(K * NWORD, 1), (K * NWORD, L))
    gml = jnp.asarray(gml, jnp.int32).reshape(K * NWORD * L)
    out3 = _make_sc_kernel()(llr3.reshape(NW, N * EPW), gml)
    out = out3.reshape(NW, N, EPW).transpose(0, 2, 1).reshape(bs, N)
    return out.reshape(shape)
